# baseline (speedup 1.0000x reference)
import functools

import jax
import jax.numpy as jnp
from jax.experimental import pallas as pl
from jax.experimental.pallas import tpu as pltpu

_BLK = 10000

_TP0 = 0.9999016017102752
_TP1 = 0.10351205418892724
_TP2 = 0.0007100632214392892
_TQ1 = 0.4365328063405299
_TQ2 = 0.01318286626827741
_CLAMP = 4.45
_MAGIC = 0x7EF311C7


def _recip(q):
    bits = jax.lax.bitcast_convert_type(q, jnp.int32)
    r = jax.lax.bitcast_convert_type(_MAGIC - bits, jnp.float32)
    r = r * (2.0 - q * r)
    r = r * (2.0 - q * r)
    return r


def _tanh(z):
    z = jnp.clip(z, -_CLAMP, _CLAMP)
    u = z * z
    p = (_TP0 + u * (_TP1 + u * _TP2)) * z
    q = 1.0 + u * (_TQ1 + u * _TQ2)
    return p * _recip(q)


def _sigmoid(z):
    return 0.5 + 0.5 * _tanh(0.5 * z)


def _dg(a, b, ca, cb):
    return jax.lax.dot_general(
        a, b, dimension_numbers=(((ca,), (cb,)), ((), ())),
        preferred_element_type=jnp.float32)


def _lstm_kernel(h_dim,
                 x_ref, h_ref, c_ref,
                 wxi_ref, wxf_ref, wxc_ref, wxo_ref,
                 whi_ref, whf_ref, whc_ref, who_ref,
                 bxi_ref, bhi_ref, bii_ref,
                 bxf_ref, bhf_ref, bff_ref,
                 bxc_ref, bhc_ref, bcc_ref,
                 bxo_ref, bho_ref, boo_ref,
                 wci_ref, wcf_ref, wco_ref, fcw_ref, fcb_ref,
                 out_ref, hn_ref, cn_ref):
    x = x_ref[...]
    h = h_ref[...]
    c = c_ref[...]

    wx = jnp.concatenate([wxi_ref[...], wxf_ref[...],
                          wxc_ref[...], wxo_ref[...]], axis=1)
    wh = jnp.concatenate([whi_ref[...], whf_ref[...],
                          whc_ref[...], who_ref[...]], axis=1)
    rr = jax.lax.broadcasted_iota(jnp.int32, (h_dim, h_dim), 0)
    cc = jax.lax.broadcasted_iota(jnp.int32, (h_dim, h_dim), 1)
    eye = (rr == cc).astype(jnp.float32)

    pre = _dg(wx, x, 0, 1) + _dg(wh, h, 0, 1)
    ct = _dg(eye, c, 1, 1)

    b_ig = bxi_ref[...] + bhi_ref[...] + bii_ref[...]
    b_fg = bxf_ref[...] + bhf_ref[...] + bff_ref[...]
    b_cg = bxc_ref[...] + bhc_ref[...] + bcc_ref[...]
    b_og = bxo_ref[...] + bho_ref[...] + boo_ref[...]

    i_g = _sigmoid(pre[0 * h_dim:1 * h_dim, :] + b_ig + wci_ref[...] * ct)
    f_g = _sigmoid(pre[1 * h_dim:2 * h_dim, :] + b_fg + wcf_ref[...] * ct)
    t_g = _tanh(pre[2 * h_dim:3 * h_dim, :] + b_cg)
    cn_t = f_g * ct + i_g * t_g
    o_g = _sigmoid(pre[3 * h_dim:4 * h_dim, :] + b_og + wco_ref[...] * cn_t)
    hn_t = o_g * _tanh(cn_t)

    cn_ref[...] = _dg(cn_t, eye, 0, 0)
    hn_ref[...] = _dg(hn_t, eye, 0, 0)
    relu_h = jnp.maximum(hn_t, 0.0)
    out_ref[...] = _dg(relu_h, fcw_ref[...], 0, 0) + fcb_ref[...]


def kernel(x, edge_index, edge_weight, h, c,
           W_xi, b_xi, W_hi, b_hi, W_xf, b_xf, W_hf, b_hf,
           W_xc, b_xc, W_hc, b_hc, W_xo, b_xo, W_ho, b_ho,
           w_ci, w_cf, w_co, b_i, b_f, b_c, b_o, fc_w, fc_b):
    del edge_index, edge_weight
    f_in = x.shape[1]
    h_dim = h.shape[1]
    n = x.shape[0]

    col = lambda v: v.reshape(h_dim, 1)
    fcb = fc_b.reshape(1, 1)

    grid = (n // _BLK,)
    row = lambda i: (i, 0)
    full = lambda i: (0, 0)
    wxs = pl.BlockSpec((f_in, h_dim), full)
    whs = pl.BlockSpec((h_dim, h_dim), full)
    cs = pl.BlockSpec((h_dim, 1), full)

    out, h_new, c_new = pl.pallas_call(
        functools.partial(_lstm_kernel, h_dim),
        grid=grid,
        in_specs=[
            pl.BlockSpec((_BLK, f_in), row),
            pl.BlockSpec((_BLK, h_dim), row),
            pl.BlockSpec((_BLK, h_dim), row),
            wxs, wxs, wxs, wxs,
            whs, whs, whs, whs,
            cs, cs, cs,
            cs, cs, cs,
            cs, cs, cs,
            cs, cs, cs,
            cs, cs, cs,
            cs,
            pl.BlockSpec((1, 1), full),
        ],
        out_specs=[
            pl.BlockSpec((_BLK, 1), row),
            pl.BlockSpec((_BLK, h_dim), row),
            pl.BlockSpec((_BLK, h_dim), row),
        ],
        out_shape=[
            jax.ShapeDtypeStruct((n, 1), jnp.float32),
            jax.ShapeDtypeStruct((n, h_dim), jnp.float32),
            jax.ShapeDtypeStruct((n, h_dim), jnp.float32),
        ],
        compiler_params=pltpu.CompilerParams(
            dimension_semantics=("arbitrary",),
        ),
    )(x, h, c,
      W_xi, W_xf, W_xc, W_xo, W_hi, W_hf, W_hc, W_ho,
      col(b_xi), col(b_hi), col(b_i),
      col(b_xf), col(b_hf), col(b_f),
      col(b_xc), col(b_hc), col(b_c),
      col(b_xo), col(b_ho), col(b_o),
      col(w_ci), col(w_cf), col(w_co), fc_w, fcb)
    return (out, h_new, c_new)


# speedup vs baseline: 1.0198x; 1.0198x over previous
import functools

import jax
import jax.numpy as jnp
from jax.experimental import pallas as pl
from jax.experimental.pallas import tpu as pltpu

_BLK = 2000

_TP0 = 0.9999016017102752
_TP1 = 0.10351205418892724
_TP2 = 0.0007100632214392892
_TQ1 = 0.4365328063405299
_TQ2 = 0.01318286626827741
_CLAMP = 4.45
_MAGIC = 0x7EF311C7


def _recip(q):
    bits = jax.lax.bitcast_convert_type(q, jnp.int32)
    r = jax.lax.bitcast_convert_type(_MAGIC - bits, jnp.float32)
    r = r * (2.0 - q * r)
    r = r * (2.0 - q * r)
    return r


def _tanh(z):
    z = jnp.clip(z, -_CLAMP, _CLAMP)
    u = z * z
    p = (_TP0 + u * (_TP1 + u * _TP2)) * z
    q = 1.0 + u * (_TQ1 + u * _TQ2)
    return p * _recip(q)


def _sigmoid(z):
    return 0.5 + 0.5 * _tanh(0.5 * z)


def _dg(a, b, ca, cb):
    return jax.lax.dot_general(
        a, b, dimension_numbers=(((ca,), (cb,)), ((), ())),
        preferred_element_type=jnp.float32)


def _assemble_kernel(h_dim,
                     wxi_ref, wxf_ref, wxc_ref, wxo_ref,
                     whi_ref, whf_ref, whc_ref, who_ref,
                     bxi_ref, bhi_ref, bii_ref,
                     bxf_ref, bhf_ref, bff_ref,
                     bxc_ref, bhc_ref, bcc_ref,
                     bxo_ref, bho_ref, boo_ref,
                     wx_ref, wh_ref, bias_ref, eye_ref):
    wx_ref[...] = jnp.concatenate(
        [wxi_ref[...], wxf_ref[...], wxc_ref[...], wxo_ref[...]], axis=1)
    wh_ref[...] = jnp.concatenate(
        [whi_ref[...], whf_ref[...], whc_ref[...], who_ref[...]], axis=1)
    bias_ref[...] = jnp.concatenate(
        [bxi_ref[...] + bhi_ref[...] + bii_ref[...],
         bxf_ref[...] + bhf_ref[...] + bff_ref[...],
         bxc_ref[...] + bhc_ref[...] + bcc_ref[...],
         bxo_ref[...] + bho_ref[...] + boo_ref[...]], axis=0)
    rr = jax.lax.broadcasted_iota(jnp.int32, (h_dim, h_dim), 0)
    cc = jax.lax.broadcasted_iota(jnp.int32, (h_dim, h_dim), 1)
    eye_ref[...] = (rr == cc).astype(jnp.float32)


def _lstm_kernel(h_dim, x_ref, h_ref, c_ref, wx_ref, wh_ref, b_ref,
                 wci_ref, wcf_ref, wco_ref, fcw_ref, fcb_ref, eye_ref,
                 out_ref, hn_ref, cn_ref):
    x = x_ref[...]
    h = h_ref[...]
    c = c_ref[...]
    eye = eye_ref[...]

    pre = _dg(wx_ref[...], x, 0, 1)
    pre = pre + _dg(wh_ref[...], h, 0, 1)
    pre = pre + b_ref[...]
    ct = _dg(eye, c, 1, 1)
    i_g = _sigmoid(pre[0 * h_dim:1 * h_dim, :] + wci_ref[...] * ct)
    f_g = _sigmoid(pre[1 * h_dim:2 * h_dim, :] + wcf_ref[...] * ct)
    t_g = _tanh(pre[2 * h_dim:3 * h_dim, :])
    cn_t = f_g * ct + i_g * t_g
    o_g = _sigmoid(pre[3 * h_dim:4 * h_dim, :] + wco_ref[...] * cn_t)
    hn_t = o_g * _tanh(cn_t)
    cn_ref[...] = _dg(cn_t, eye, 0, 0)
    hn_ref[...] = _dg(hn_t, eye, 0, 0)
    relu_h = jnp.maximum(hn_t, 0.0)
    out_ref[...] = _dg(relu_h, fcw_ref[...], 0, 0) + fcb_ref[...]


def kernel(x, edge_index, edge_weight, h, c,
           W_xi, b_xi, W_hi, b_hi, W_xf, b_xf, W_hf, b_hf,
           W_xc, b_xc, W_hc, b_hc, W_xo, b_xo, W_ho, b_ho,
           w_ci, w_cf, w_co, b_i, b_f, b_c, b_o, fc_w, fc_b):
    del edge_index, edge_weight
    f_in = x.shape[1]
    h_dim = h.shape[1]
    n = x.shape[0]

    col = lambda v: v.reshape(h_dim, 1)
    fcb = fc_b.reshape(1, 1)

    one = lambda i: (0, 0)
    wxs = pl.BlockSpec((f_in, h_dim), one)
    whs = pl.BlockSpec((h_dim, h_dim), one)
    cs = pl.BlockSpec((h_dim, 1), one)
    wx, wh, bias, eye = pl.pallas_call(
        functools.partial(_assemble_kernel, h_dim),
        grid=(1,),
        in_specs=[wxs, wxs, wxs, wxs, whs, whs, whs, whs,
                  cs, cs, cs, cs, cs, cs, cs, cs, cs, cs, cs, cs],
        out_specs=[pl.BlockSpec((f_in, 4 * h_dim), one),
                   pl.BlockSpec((h_dim, 4 * h_dim), one),
                   pl.BlockSpec((4 * h_dim, 1), one),
                   pl.BlockSpec((h_dim, h_dim), one)],
        out_shape=[jax.ShapeDtypeStruct((f_in, 4 * h_dim), jnp.float32),
                   jax.ShapeDtypeStruct((h_dim, 4 * h_dim), jnp.float32),
                   jax.ShapeDtypeStruct((4 * h_dim, 1), jnp.float32),
                   jax.ShapeDtypeStruct((h_dim, h_dim), jnp.float32)],
    )(W_xi, W_xf, W_xc, W_xo, W_hi, W_hf, W_hc, W_ho,
      col(b_xi), col(b_hi), col(b_i),
      col(b_xf), col(b_hf), col(b_f),
      col(b_xc), col(b_hc), col(b_c),
      col(b_xo), col(b_ho), col(b_o))

    grid = (n // _BLK,)
    row = lambda i: (i, 0)
    out, h_new, c_new = pl.pallas_call(
        functools.partial(_lstm_kernel, h_dim),
        grid=grid,
        in_specs=[
            pl.BlockSpec((_BLK, f_in), row),
            pl.BlockSpec((_BLK, h_dim), row),
            pl.BlockSpec((_BLK, h_dim), row),
            pl.BlockSpec((f_in, 4 * h_dim), one),
            pl.BlockSpec((h_dim, 4 * h_dim), one),
            pl.BlockSpec((4 * h_dim, 1), one),
            cs, cs, cs,
            cs,
            pl.BlockSpec((1, 1), one),
            pl.BlockSpec((h_dim, h_dim), one),
        ],
        out_specs=[
            pl.BlockSpec((_BLK, 1), row),
            pl.BlockSpec((_BLK, h_dim), row),
            pl.BlockSpec((_BLK, h_dim), row),
        ],
        out_shape=[
            jax.ShapeDtypeStruct((n, 1), jnp.float32),
            jax.ShapeDtypeStruct((n, h_dim), jnp.float32),
            jax.ShapeDtypeStruct((n, h_dim), jnp.float32),
        ],
        compiler_params=pltpu.CompilerParams(
            dimension_semantics=("arbitrary",),
        ),
    )(x, h, c, wx, wh, bias,
      col(w_ci), col(w_cf), col(w_co), fc_w, fcb, eye)
    return (out, h_new, c_new)


# speedup vs baseline: 1.3722x; 1.3456x over previous
import jax
import jax.numpy as jnp
from jax.experimental import pallas as pl
from jax.experimental.pallas import tpu as pltpu

_BLK = 1000


def _lstm_kernel(x_ref, h_ref, c_ref, wx_ref, wh_ref, b_ref,
                 wci_ref, wcf_ref, wco_ref, fcw_ref, fcb_ref,
                 out_ref, hn_ref, cn_ref):
    x = x_ref[...]
    h = h_ref[...]
    c = c_ref[...]
    pre = jnp.dot(x, wx_ref[...], preferred_element_type=jnp.float32)
    pre = pre + jnp.dot(h, wh_ref[...], preferred_element_type=jnp.float32)
    pre = pre + b_ref[...]
    i_g = jax.nn.sigmoid(pre[:, 0:32] + wci_ref[...] * c)
    f_g = jax.nn.sigmoid(pre[:, 32:64] + wcf_ref[...] * c)
    t_g = jnp.tanh(pre[:, 64:96])
    c_new = f_g * c + i_g * t_g
    o_g = jax.nn.sigmoid(pre[:, 96:128] + wco_ref[...] * c_new)
    h_new = o_g * jnp.tanh(c_new)
    cn_ref[...] = c_new
    hn_ref[...] = h_new
    relu_h = jnp.maximum(h_new, 0.0)
    out_ref[...] = (jnp.sum(relu_h * fcw_ref[...], axis=1, keepdims=True)
                    + fcb_ref[...])


def kernel(x, edge_index, edge_weight, h, c,
           W_xi, b_xi, W_hi, b_hi, W_xf, b_xf, W_hf, b_hf,
           W_xc, b_xc, W_hc, b_hc, W_xo, b_xo, W_ho, b_ho,
           w_ci, w_cf, w_co, b_i, b_f, b_c, b_o, fc_w, fc_b):
    del edge_index, edge_weight
    f_in = x.shape[1]
    h_dim = h.shape[1]
    wx = jnp.concatenate([W_xi, W_xf, W_xc, W_xo], axis=1)
    wh = jnp.concatenate([W_hi, W_hf, W_hc, W_ho], axis=1)
    bias = jnp.concatenate([b_xi + b_hi + b_i[0],
                            b_xf + b_hf + b_f[0],
                            b_xc + b_hc + b_c[0],
                            b_xo + b_ho + b_o[0]])[None, :]
    fcw = fc_w.T
    fcb = fc_b.reshape(1, 1)

    n = x.shape[0]
    grid = (n // _BLK,)
    row = lambda i: (i, 0)
    full = lambda i: (0, 0)
    out, h_new, c_new = pl.pallas_call(
        _lstm_kernel,
        grid=grid,
        in_specs=[
            pl.BlockSpec((_BLK, f_in), row),
            pl.BlockSpec((_BLK, h_dim), row),
            pl.BlockSpec((_BLK, h_dim), row),
            pl.BlockSpec((f_in, 4 * h_dim), full),
            pl.BlockSpec((h_dim, 4 * h_dim), full),
            pl.BlockSpec((1, 4 * h_dim), full),
            pl.BlockSpec((1, h_dim), full),
            pl.BlockSpec((1, h_dim), full),
            pl.BlockSpec((1, h_dim), full),
            pl.BlockSpec((1, h_dim), full),
            pl.BlockSpec((1, 1), full),
        ],
        out_specs=[
            pl.BlockSpec((_BLK, 1), row),
            pl.BlockSpec((_BLK, h_dim), row),
            pl.BlockSpec((_BLK, h_dim), row),
        ],
        out_shape=[
            jax.ShapeDtypeStruct((n, 1), jnp.float32),
            jax.ShapeDtypeStruct((n, h_dim), jnp.float32),
            jax.ShapeDtypeStruct((n, h_dim), jnp.float32),
        ],
        compiler_params=pltpu.CompilerParams(
            dimension_semantics=("arbitrary",),
        ),
    )(x, h, c, wx, wh, bias, w_ci, w_cf, w_co, fcw, fcb)
    return (out, h_new, c_new)
